# baseline (speedup 1.0000x reference)
import jax
import jax.numpy as jnp
from jax import lax
from jax.experimental import pallas as pl
from jax.experimental.pallas import tpu as pltpu
from jax.experimental.pallas import tpu_sc as plsc

N = 10000
E = 320000
NC = 2
NS = 16
NW = NC * NS
CHUNK = 128
CPT = 80
EPT = CHUNK * CPT
EPAD = NW * EPT
NPT = 640
NPAD = NS * NPT
BLK = 4096

_MESH = plsc.VectorSubcoreMesh(core_axis_name="c", subcore_axis_name="s")
_SC_PARAMS = pltpu.CompilerParams(use_tc_tiling_on_sc=False)


def _worker(c, s):
    return c * NS + s


def _deg_body(dst_hbm, ones_hbm, zeros_hbm, out_hbm, idx_v, ones_v, deg_sh,
              s0, s1, s2, s3, s4, s5, s6, s7):
    c = lax.axis_index("c")
    s = lax.axis_index("s")
    w = _worker(c, s)
    sems = (s0, s1, s2, s3, s4, s5, s6, s7)
    pltpu.sync_copy(ones_hbm, ones_v)
    pltpu.sync_copy(dst_hbm.at[pl.ds(w * CPT, CPT)], idx_v)
    pltpu.sync_copy(zeros_hbm, deg_sh.at[pl.ds(s * NPT, NPT)])
    plsc.subcore_barrier()
    for k in range(CPT):
        m = k % 8
        if k >= 8:
            pltpu.make_async_copy(ones_v, deg_sh.at[idx_v.at[k - 8]],
                                  sems[m]).wait()
        pltpu.async_copy(ones_v, deg_sh.at[idx_v.at[k]], sems[m], add=True)
    for k in range(CPT - 8, CPT):
        m = k % 8
        pltpu.make_async_copy(ones_v, deg_sh.at[idx_v.at[k]], sems[m]).wait()
    plsc.subcore_barrier()
    pltpu.sync_copy(deg_sh.at[pl.ds(s * NPT, NPT)],
                    out_hbm.at[c, pl.ds(s * NPT, NPT)])


def _deg_counts(dst2d):
    fn = pl.kernel(
        _deg_body,
        out_type=jax.ShapeDtypeStruct((NC, NPAD, 8), jnp.float32),
        mesh=_MESH,
        compiler_params=_SC_PARAMS,
        scratch_types=[
            pltpu.VMEM((CPT, CHUNK), jnp.int32),
            pltpu.VMEM((CHUNK, 8), jnp.float32),
            pltpu.VMEM_SHARED((NPAD, 8), jnp.float32),
        ] + [pltpu.SemaphoreType.DMA] * 8,
    )
    return fn(dst2d, jnp.ones((CHUNK, 8), jnp.float32),
              jnp.zeros((NPT, 8), jnp.float32))


def _agg_body(*refs):
    (src_hbm, dst_hbm, tab_hbm, zeros_hbm, out_hbm, idxs_v, idxd_v) = refs[:7]
    rows = refs[7:15]
    agg_sh = refs[15]
    g = refs[16:24]
    sc = refs[24:32]
    c = lax.axis_index("c")
    s = lax.axis_index("s")
    w = _worker(c, s)
    pltpu.sync_copy(src_hbm.at[pl.ds(w * CPT, CPT)], idxs_v)
    pltpu.sync_copy(dst_hbm.at[pl.ds(w * CPT, CPT)], idxd_v)
    pltpu.sync_copy(zeros_hbm, agg_sh.at[pl.ds(s * NPT, NPT)])
    plsc.subcore_barrier()

    def fire_gather(k, b):
        pltpu.async_copy(tab_hbm.at[idxs_v.at[k]], rows[b], g[b])

    def wait_gather(k, b):
        pltpu.make_async_copy(tab_hbm.at[idxs_v.at[k]], rows[b], g[b]).wait()

    def fire_scatter(k, b):
        pltpu.async_copy(rows[b], agg_sh.at[idxd_v.at[k]], sc[b], add=True)

    def wait_scatter(k, b):
        pltpu.make_async_copy(rows[b], agg_sh.at[idxd_v.at[k]], sc[b]).wait()

    for k in range(4):
        fire_gather(k, k)
    for k in range(4):
        wait_gather(k, k)
        fire_scatter(k, k)
        fire_gather(k + 4, k + 4)

    def step(j, carry):
        for m in range(8):
            k = 8 * j + 4 + m
            b = (4 + m) % 8
            wait_gather(k, b)
            fire_scatter(k, b)
            wait_scatter(k - 4, m)
            fire_gather(k + 4, m)
        return carry

    lax.fori_loop(0, (CPT - 8) // 8, step, 0)

    for k in range(CPT - 4, CPT):
        b = k % 8
        wait_gather(k, b)
        fire_scatter(k, b)
    for k in range(CPT - 8, CPT):
        wait_scatter(k, k % 8)
    plsc.subcore_barrier()
    pltpu.sync_copy(agg_sh.at[pl.ds(s * NPT, NPT)],
                    out_hbm.at[c, pl.ds(s * NPT, NPT)])


def _edge_agg(src2d, dst2d, table, width):
    fn = pl.kernel(
        _agg_body,
        out_type=jax.ShapeDtypeStruct((NC, NPAD, width), jnp.float32),
        mesh=_MESH,
        compiler_params=_SC_PARAMS,
        scratch_types=[
            pltpu.VMEM((CPT, CHUNK), jnp.int32),
            pltpu.VMEM((CPT, CHUNK), jnp.int32),
        ] + [pltpu.VMEM((CHUNK, width), jnp.float32)] * 8 + [
            pltpu.VMEM_SHARED((NPAD, width), jnp.float32),
        ] + [pltpu.SemaphoreType.DMA] * 16,
    )
    return fn(src2d, dst2d, table, jnp.zeros((NPT, width), jnp.float32))


def _pair_body(*refs):
    (src_hbm, dst_hbm, gs_hbm, gd_hbm, out_hbm, idxs_v, idxd_v) = refs[:7]
    av = refs[7:11]
    bv = refs[11:15]
    cv = refs[15:19]
    ga = refs[19:23]
    gb = refs[23:27]
    st = refs[27:31]
    c = lax.axis_index("c")
    s = lax.axis_index("s")
    w = _worker(c, s)
    rbase = w * (EPT // 4)
    pltpu.sync_copy(src_hbm.at[pl.ds(w * CPT, CPT)], idxs_v)
    pltpu.sync_copy(dst_hbm.at[pl.ds(w * CPT, CPT)], idxd_v)

    def fire_gathers(k, b):
        pltpu.async_copy(gs_hbm.at[idxs_v.at[k]], av[b], ga[b])
        pltpu.async_copy(gd_hbm.at[idxd_v.at[k]], bv[b], gb[b])

    def wait_gathers(k, b):
        pltpu.make_async_copy(gs_hbm.at[idxs_v.at[k]], av[b], ga[b]).wait()
        pltpu.make_async_copy(gd_hbm.at[idxd_v.at[k]], bv[b], gb[b]).wait()

    def do_adds(b):
        for i in range(CHUNK):
            for h in (0, 16):
                pos = 32 * i + h
                cv[b][pos // 128, pl.ds(pos % 128, 16)] = (
                    av[b][i, pl.ds(h, 16)] + bv[b][i, pl.ds(h, 16)])

    def fire_store(k, b):
        pltpu.async_copy(cv[b], out_hbm.at[pl.ds(rbase + k * 32, 32)], st[b])

    def wait_store(k, b):
        pltpu.make_async_copy(cv[b], out_hbm.at[pl.ds(rbase + k * 32, 32)],
                              st[b]).wait()

    fire_gathers(0, 0)
    fire_gathers(1, 1)
    for k in (0, 1):
        wait_gathers(k, k)
        fire_gathers(k + 2, k + 2)
        do_adds(k)
        fire_store(k, k)

    def step(j, carry):
        for m in range(4):
            k = 4 * j + 2 + m
            b = (2 + m) % 4
            wait_gathers(k, b)
            wait_store(k - 2, m)
            fire_gathers(k + 2, m)
            do_adds(b)
            fire_store(k, b)
        return carry

    lax.fori_loop(0, (CPT - 4) // 4, step, 0)

    for k in (CPT - 2, CPT - 1):
        b = k % 4
        wait_gathers(k, b)
        wait_store(k - 2, (k + 2) % 4)
        do_adds(b)
        fire_store(k, b)
    for k in range(CPT - 2, CPT):
        wait_store(k, k % 4)


def _pair_sums(src2d, dst2d, gs, gd):
    fn = pl.kernel(
        _pair_body,
        out_type=jax.ShapeDtypeStruct((EPAD // 4, 128), jnp.float32),
        mesh=_MESH,
        compiler_params=_SC_PARAMS,
        scratch_types=[
            pltpu.VMEM((CPT, CHUNK), jnp.int32),
            pltpu.VMEM((CPT, CHUNK), jnp.int32),
        ] + [pltpu.VMEM((CHUNK, 32), jnp.float32)] * 8
          + [pltpu.VMEM((32, 128), jnp.float32)] * 4
          + [pltpu.SemaphoreType.DMA] * 12,
    )
    return fn(src2d, dst2d, gs, gd)


def _tc1_body(x_ref, w1_ref, degp_ref, hp_ref, dinv_ref):
    deg = degp_ref[0] + degp_ref[1] + 1.0
    dinv = lax.rsqrt(jnp.maximum(deg, 1e-12))
    dinv_ref[...] = dinv
    h = jnp.dot(x_ref[...], w1_ref[...], preferred_element_type=jnp.float32)
    hp_ref[...] = h * dinv[:, 0:1]


def _tc2_body(aggp_ref, hp_ref, dinv_ref, b1_ref, w2_ref, hp2_ref):
    dinv = dinv_ref[:, 0:1]
    h1 = jnp.maximum((aggp_ref[0] + aggp_ref[1] + hp_ref[...]) * dinv
                     + b1_ref[...], 0.0)
    h2 = jnp.dot(h1, w2_ref[...], preferred_element_type=jnp.float32)
    hp2_ref[...] = h2 * dinv


def _tc3_body(aggp_ref, hp2_ref, dinv_ref, b2_ref, w3a_ref, w3b_ref,
              gs_ref, gd_ref):
    dinv = dinv_ref[:, 0:1]
    h2 = jnp.maximum((aggp_ref[0] + aggp_ref[1] + hp2_ref[...]) * dinv
                     + b2_ref[...], 0.0)
    gs_ref[...] = jnp.dot(h2, w3a_ref[...],
                          preferred_element_type=jnp.float32)
    gd_ref[...] = jnp.dot(h2, w3b_ref[...],
                          preferred_element_type=jnp.float32)


def _attr_body(ar_ref, wg_ref, b3t_ref, a2_ref):
    nr = a2_ref.shape[0]
    nq = nr // 8
    ar = ar_ref[...]
    rid = jax.lax.broadcasted_iota(jnp.int32, (nr, 128), 0) % 8
    acc = b3t_ref[...] + jnp.zeros((nr, 128), jnp.float32)
    for g in range(8):
        p = jnp.dot(ar, wg_ref[g], preferred_element_type=jnp.float32)
        pe = jnp.broadcast_to(p[:, None, :], (nq, 8, 128)).reshape(nr, 128)
        acc = acc + jnp.where(rid == g, pe, 0.0)
    a2_ref[...] = acc


def _tc4_body(p_ref, a2_ref, wbig_ref, b4_ref, out_ref):
    h = jnp.maximum(p_ref[...] + a2_ref[...], 0.0)
    t = jnp.dot(h, wbig_ref[...], preferred_element_type=jnp.float32)
    nr = out_ref.shape[0]
    t3 = t.reshape(nr, 32, 128)
    lane_q = jax.lax.broadcasted_iota(jnp.int32, (nr, 128), 1) // 4
    acc = b4_ref[...] + jnp.zeros((nr, 128), jnp.float32)
    for q in range(32):
        acc = acc + jnp.where(lane_q == q, t3[:, q, :], 0.0)
    out_ref[...] = acc


def kernel(x, edge_index, edge_attr, W1, b1, W2, b2, W3, b3, W4, b4):
    src = edge_index[0].astype(jnp.int32)
    dst = edge_index[1].astype(jnp.int32)
    pad_e = EPAD - E
    src2d = jnp.concatenate(
        [src, jnp.full((pad_e,), N, jnp.int32)]).reshape(NW * CPT, CHUNK)
    dst2d = jnp.concatenate(
        [dst, jnp.full((pad_e,), N, jnp.int32)]).reshape(NW * CPT, CHUNK)
    x_pad = jnp.pad(x, ((0, NPAD - N), (0, 0)))

    ar128 = jnp.pad(edge_attr.reshape(E // 32, 128),
                    ((0, (EPAD - E) // 32), (0, 0)))
    w3c = W3[32:36]
    wg = jnp.zeros((8, 128, 128), jnp.float32)
    for g in range(8):
        for m in range(4):
            for t in range(4):
                wg = wg.at[g, 16 * g + 4 * m + t, 32 * m:32 * m + 32].set(w3c[t])
    a2 = pl.pallas_call(
        _attr_body,
        grid=(40,),
        in_specs=[
            pl.BlockSpec((256, 128), lambda i: (i, 0)),
            pl.BlockSpec((8, 128, 128), lambda i: (0, 0, 0)),
            pl.BlockSpec((1, 128), lambda i: (0, 0)),
        ],
        out_specs=pl.BlockSpec((2048, 128), lambda i: (i, 0)),
        out_shape=jax.ShapeDtypeStruct((EPAD // 4, 128), jnp.float32),
    )(ar128, wg, jnp.tile(b3, 4)[None, :])

    degp = _deg_counts(dst2d)

    hp1, dinv8 = pl.pallas_call(
        _tc1_body,
        out_shape=[jax.ShapeDtypeStruct((NPAD, 32), jnp.float32),
                   jax.ShapeDtypeStruct((NPAD, 8), jnp.float32)],
    )(x_pad, W1, degp)

    aggp1 = _edge_agg(src2d, dst2d, hp1, 32)

    hp2 = pl.pallas_call(
        _tc2_body,
        out_shape=jax.ShapeDtypeStruct((NPAD, 16), jnp.float32),
    )(aggp1, hp1, dinv8, b1[None, :], W2)

    aggp2 = _edge_agg(src2d, dst2d, hp2, 16)

    gs, gd = pl.pallas_call(
        _tc3_body,
        out_shape=[jax.ShapeDtypeStruct((NPAD, 32), jnp.float32),
                   jax.ShapeDtypeStruct((NPAD, 32), jnp.float32)],
    )(aggp2, hp2, dinv8, b2[None, :], W3[0:16], W3[16:32])

    p_sum = _pair_sums(src2d, dst2d, gs, gd)

    blk_r = 2048
    out2d = pl.pallas_call(
        _tc4_body,
        grid=(EPAD // (4 * blk_r),),
        in_specs=[
            pl.BlockSpec((blk_r, 128), lambda i: (i, 0)),
            pl.BlockSpec((blk_r, 128), lambda i: (i, 0)),
            pl.BlockSpec((128, 128), lambda i: (0, 0)),
            pl.BlockSpec((1, 128), lambda i: (0, 0)),
        ],
        out_specs=pl.BlockSpec((blk_r // 32, 128), lambda i: (i, 0)),
        out_shape=jax.ShapeDtypeStruct((EPAD // 128, 128), jnp.float32),
    )(p_sum, a2,
      jnp.kron(jnp.eye(4, dtype=jnp.float32), W4)
      @ (jnp.arange(128)[None, :] % 4
         == jnp.arange(4)[:, None]).astype(jnp.float32),
      jnp.broadcast_to(b4, (1, 128)))

    return out2d[:E // 128].reshape(E)


# speedup vs baseline: 1.3163x; 1.3163x over previous
import jax
import jax.numpy as jnp
from jax import lax
from jax.experimental import pallas as pl
from jax.experimental.pallas import tpu as pltpu
from jax.experimental.pallas import tpu_sc as plsc

N = 10000
E = 320000
NC = 2
NS = 16
NW = NC * NS
CHUNK = 128
CPT = 80
EPT = CHUNK * CPT
EPAD = NW * EPT
NPT = 640
NPAD = NS * NPT
BLK = 4096

_MESH = plsc.VectorSubcoreMesh(core_axis_name="c", subcore_axis_name="s")
_SC_PARAMS = pltpu.CompilerParams(use_tc_tiling_on_sc=False)


def _worker(c, s):
    return c * NS + s


def _deg_body(dst_hbm, ones_hbm, zeros_hbm, out_hbm, idx_v, ones_v, deg_sh,
              s0, s1, s2, s3, s4, s5, s6, s7):
    c = lax.axis_index("c")
    s = lax.axis_index("s")
    w = _worker(c, s)
    sems = (s0, s1, s2, s3, s4, s5, s6, s7)
    pltpu.sync_copy(ones_hbm, ones_v)
    pltpu.sync_copy(dst_hbm.at[pl.ds(w * CPT, CPT)], idx_v)
    pltpu.sync_copy(zeros_hbm, deg_sh.at[pl.ds(s * NPT, NPT)])
    plsc.subcore_barrier()
    for k in range(CPT):
        m = k % 8
        if k >= 8:
            pltpu.make_async_copy(ones_v, deg_sh.at[idx_v.at[k - 8]],
                                  sems[m]).wait()
        pltpu.async_copy(ones_v, deg_sh.at[idx_v.at[k]], sems[m], add=True)
    for k in range(CPT - 8, CPT):
        m = k % 8
        pltpu.make_async_copy(ones_v, deg_sh.at[idx_v.at[k]], sems[m]).wait()
    plsc.subcore_barrier()
    pltpu.sync_copy(deg_sh.at[pl.ds(s * NPT, NPT)],
                    out_hbm.at[c, pl.ds(s * NPT, NPT)])


def _deg_counts(dst2d):
    fn = pl.kernel(
        _deg_body,
        out_type=jax.ShapeDtypeStruct((NC, NPAD, 8), jnp.float32),
        mesh=_MESH,
        compiler_params=_SC_PARAMS,
        scratch_types=[
            pltpu.VMEM((CPT, CHUNK), jnp.int32),
            pltpu.VMEM((CHUNK, 8), jnp.float32),
            pltpu.VMEM_SHARED((NPAD, 8), jnp.float32),
        ] + [pltpu.SemaphoreType.DMA] * 8,
    )
    return fn(dst2d, jnp.ones((CHUNK, 8), jnp.float32),
              jnp.zeros((NPT, 8), jnp.float32))


def _agg_body(*refs):
    (src_hbm, dst_hbm, tab_hbm, zeros_hbm, out_hbm, idxs_v, idxd_v) = refs[:7]
    rows = refs[7:15]
    agg_sh = refs[15]
    g = refs[16:24]
    sc = refs[24:32]
    c = lax.axis_index("c")
    s = lax.axis_index("s")
    w = _worker(c, s)
    pltpu.sync_copy(src_hbm.at[pl.ds(w * CPT, CPT)], idxs_v)
    pltpu.sync_copy(dst_hbm.at[pl.ds(w * CPT, CPT)], idxd_v)
    pltpu.sync_copy(zeros_hbm, agg_sh.at[pl.ds(s * NPT, NPT)])
    plsc.subcore_barrier()

    def fire_gather(k, b):
        pltpu.async_copy(tab_hbm.at[idxs_v.at[k]], rows[b], g[b])

    def wait_gather(k, b):
        pltpu.make_async_copy(tab_hbm.at[idxs_v.at[k]], rows[b], g[b]).wait()

    def fire_scatter(k, b):
        pltpu.async_copy(rows[b], agg_sh.at[idxd_v.at[k]], sc[b], add=True)

    def wait_scatter(k, b):
        pltpu.make_async_copy(rows[b], agg_sh.at[idxd_v.at[k]], sc[b]).wait()

    for k in range(4):
        fire_gather(k, k)
    for k in range(4):
        wait_gather(k, k)
        fire_scatter(k, k)
        fire_gather(k + 4, k + 4)

    def step(j, carry):
        for m in range(8):
            k = 8 * j + 4 + m
            b = (4 + m) % 8
            wait_gather(k, b)
            fire_scatter(k, b)
            wait_scatter(k - 4, m)
            fire_gather(k + 4, m)
        return carry

    lax.fori_loop(0, (CPT - 8) // 8, step, 0)

    for k in range(CPT - 4, CPT):
        b = k % 8
        wait_gather(k, b)
        fire_scatter(k, b)
    for k in range(CPT - 8, CPT):
        wait_scatter(k, k % 8)
    plsc.subcore_barrier()
    pltpu.sync_copy(agg_sh.at[pl.ds(s * NPT, NPT)],
                    out_hbm.at[c, pl.ds(s * NPT, NPT)])


def _edge_agg(src2d, dst2d, table, width):
    fn = pl.kernel(
        _agg_body,
        out_type=jax.ShapeDtypeStruct((NC, NPAD, width), jnp.float32),
        mesh=_MESH,
        compiler_params=_SC_PARAMS,
        scratch_types=[
            pltpu.VMEM((CPT, CHUNK), jnp.int32),
            pltpu.VMEM((CPT, CHUNK), jnp.int32),
        ] + [pltpu.VMEM((CHUNK, width), jnp.float32)] * 8 + [
            pltpu.VMEM_SHARED((NPAD, width), jnp.float32),
        ] + [pltpu.SemaphoreType.DMA] * 16,
    )
    return fn(src2d, dst2d, table, jnp.zeros((NPT, width), jnp.float32))


def _pair_body(*refs):
    (src_hbm, dst_hbm, gs_hbm, gd_hbm, out_hbm, idxs_v, idxd_v) = refs[:7]
    av = refs[7:11]
    bv = refs[11:15]
    cv = refs[15:19]
    ga = refs[19:23]
    gb = refs[23:27]
    st = refs[27:31]
    c = lax.axis_index("c")
    s = lax.axis_index("s")
    w = _worker(c, s)
    rbase = w * (EPT // 4)
    pltpu.sync_copy(src_hbm.at[pl.ds(w * CPT, CPT)], idxs_v)
    pltpu.sync_copy(dst_hbm.at[pl.ds(w * CPT, CPT)], idxd_v)

    def fire_gathers(k, b):
        pltpu.async_copy(gs_hbm.at[idxs_v.at[k]], av[b], ga[b])
        pltpu.async_copy(gd_hbm.at[idxd_v.at[k]], bv[b], gb[b])

    def wait_gathers(k, b):
        pltpu.make_async_copy(gs_hbm.at[idxs_v.at[k]], av[b], ga[b]).wait()
        pltpu.make_async_copy(gd_hbm.at[idxd_v.at[k]], bv[b], gb[b]).wait()

    def do_adds(b):
        for i in range(CHUNK):
            for h in (0, 16):
                pos = 32 * i + h
                cv[b][pos // 128, pl.ds(pos % 128, 16)] = (
                    av[b][i, pl.ds(h, 16)] + bv[b][i, pl.ds(h, 16)])

    def fire_store(k, b):
        pltpu.async_copy(cv[b], out_hbm.at[pl.ds(rbase + k * 32, 32)], st[b])

    def wait_store(k, b):
        pltpu.make_async_copy(cv[b], out_hbm.at[pl.ds(rbase + k * 32, 32)],
                              st[b]).wait()

    fire_gathers(0, 0)
    fire_gathers(1, 1)
    for k in (0, 1):
        wait_gathers(k, k)
        fire_gathers(k + 2, k + 2)
        do_adds(k)
        fire_store(k, k)

    def step(j, carry):
        for m in range(4):
            k = 4 * j + 2 + m
            b = (2 + m) % 4
            wait_gathers(k, b)
            wait_store(k - 2, m)
            fire_gathers(k + 2, m)
            do_adds(b)
            fire_store(k, b)
        return carry

    lax.fori_loop(0, (CPT - 4) // 4, step, 0)

    for k in (CPT - 2, CPT - 1):
        b = k % 4
        wait_gathers(k, b)
        wait_store(k - 2, (k + 2) % 4)
        do_adds(b)
        fire_store(k, b)
    for k in range(CPT - 2, CPT):
        wait_store(k, k % 4)


def _pair_sums(src2d, dst2d, gs, gd):
    fn = pl.kernel(
        _pair_body,
        out_type=jax.ShapeDtypeStruct((EPAD // 4, 128), jnp.float32),
        mesh=_MESH,
        compiler_params=_SC_PARAMS,
        scratch_types=[
            pltpu.VMEM((CPT, CHUNK), jnp.int32),
            pltpu.VMEM((CPT, CHUNK), jnp.int32),
        ] + [pltpu.VMEM((CHUNK, 32), jnp.float32)] * 8
          + [pltpu.VMEM((32, 128), jnp.float32)] * 4
          + [pltpu.SemaphoreType.DMA] * 12,
    )
    return fn(src2d, dst2d, gs, gd)


def _tc1_body(x_ref, w1_ref, degp_ref, hp_ref, dinv_ref):
    deg = degp_ref[0] + degp_ref[1] + 1.0
    dinv = lax.rsqrt(jnp.maximum(deg, 1e-12))
    dinv_ref[...] = dinv
    h = jnp.dot(x_ref[...], w1_ref[...], preferred_element_type=jnp.float32)
    hp_ref[...] = h * dinv[:, 0:1]


def _tc2_body(aggp_ref, hp_ref, dinv_ref, b1_ref, w2_ref, hp2_ref):
    dinv = dinv_ref[:, 0:1]
    h1 = jnp.maximum((aggp_ref[0] + aggp_ref[1] + hp_ref[...]) * dinv
                     + b1_ref[...], 0.0)
    h2 = jnp.dot(h1, w2_ref[...], preferred_element_type=jnp.float32)
    hp2_ref[...] = h2 * dinv


def _tc3_body(aggp_ref, hp2_ref, dinv_ref, b2_ref, w3a_ref, w3b_ref,
              gs_ref, gd_ref):
    dinv = dinv_ref[:, 0:1]
    h2 = jnp.maximum((aggp_ref[0] + aggp_ref[1] + hp2_ref[...]) * dinv
                     + b2_ref[...], 0.0)
    gs_ref[...] = jnp.dot(h2, w3a_ref[...],
                          preferred_element_type=jnp.float32)
    gd_ref[...] = jnp.dot(h2, w3b_ref[...],
                          preferred_element_type=jnp.float32)


def _attr_body(attr_ref, w_ref, b3t_ref, a2_ref):
    nr = a2_ref.shape[0]
    a3 = attr_ref[...].reshape(nr, 4, 4)
    acc = b3t_ref[...] + jnp.zeros((nr, 128), jnp.float32)
    for m in range(4):
        acc = acc + jnp.dot(a3[:, m, :], w_ref[m],
                            preferred_element_type=jnp.float32)
    a2_ref[...] = acc


def _tc4_body(p_ref, a2_ref, wbig_ref, b4_ref, out_ref):
    h = jnp.maximum(p_ref[...] + a2_ref[...], 0.0)
    t = jnp.dot(h, wbig_ref[...], preferred_element_type=jnp.float32)
    nr = out_ref.shape[0]
    t3 = t.reshape(nr, 32, 128)
    lane_q = jax.lax.broadcasted_iota(jnp.int32, (nr, 128), 1) // 4
    acc = b4_ref[...] + jnp.zeros((nr, 128), jnp.float32)
    for q in range(32):
        acc = acc + jnp.where(lane_q == q, t3[:, q, :], 0.0)
    out_ref[...] = acc


def kernel(x, edge_index, edge_attr, W1, b1, W2, b2, W3, b3, W4, b4):
    src = edge_index[0].astype(jnp.int32)
    dst = edge_index[1].astype(jnp.int32)
    pad_e = EPAD - E
    src2d = jnp.concatenate(
        [src, jnp.full((pad_e,), N, jnp.int32)]).reshape(NW * CPT, CHUNK)
    dst2d = jnp.concatenate(
        [dst, jnp.full((pad_e,), N, jnp.int32)]).reshape(NW * CPT, CHUNK)
    x_pad = jnp.pad(x, ((0, NPAD - N), (0, 0)))

    w3c_sh = jnp.zeros((4, 4, 128), jnp.float32)
    for m in range(4):
        w3c_sh = w3c_sh.at[m, :, 32 * m:32 * m + 32].set(W3[32:36])
    a2 = pl.pallas_call(
        _attr_body,
        grid=(40,),
        in_specs=[
            pl.BlockSpec((8192, 4), lambda i: (i, 0)),
            pl.BlockSpec((4, 4, 128), lambda i: (0, 0, 0)),
            pl.BlockSpec((1, 128), lambda i: (0, 0)),
        ],
        out_specs=pl.BlockSpec((2048, 128), lambda i: (i, 0)),
        out_shape=jax.ShapeDtypeStruct((EPAD // 4, 128), jnp.float32),
    )(edge_attr, w3c_sh, jnp.tile(b3, 4)[None, :])

    degp = _deg_counts(dst2d)

    hp1, dinv8 = pl.pallas_call(
        _tc1_body,
        out_shape=[jax.ShapeDtypeStruct((NPAD, 32), jnp.float32),
                   jax.ShapeDtypeStruct((NPAD, 8), jnp.float32)],
    )(x_pad, W1, degp)

    aggp1 = _edge_agg(src2d, dst2d, hp1, 32)

    hp2 = pl.pallas_call(
        _tc2_body,
        out_shape=jax.ShapeDtypeStruct((NPAD, 16), jnp.float32),
    )(aggp1, hp1, dinv8, b1[None, :], W2)

    aggp2 = _edge_agg(src2d, dst2d, hp2, 16)

    gs, gd = pl.pallas_call(
        _tc3_body,
        out_shape=[jax.ShapeDtypeStruct((NPAD, 32), jnp.float32),
                   jax.ShapeDtypeStruct((NPAD, 32), jnp.float32)],
    )(aggp2, hp2, dinv8, b2[None, :], W3[0:16], W3[16:32])

    p_sum = _pair_sums(src2d, dst2d, gs, gd)

    blk_r = 2048
    out2d = pl.pallas_call(
        _tc4_body,
        grid=(EPAD // (4 * blk_r),),
        in_specs=[
            pl.BlockSpec((blk_r, 128), lambda i: (i, 0)),
            pl.BlockSpec((blk_r, 128), lambda i: (i, 0)),
            pl.BlockSpec((128, 128), lambda i: (0, 0)),
            pl.BlockSpec((1, 128), lambda i: (0, 0)),
        ],
        out_specs=pl.BlockSpec((blk_r // 32, 128), lambda i: (i, 0)),
        out_shape=jax.ShapeDtypeStruct((EPAD // 128, 128), jnp.float32),
    )(p_sum, a2,
      jnp.kron(jnp.eye(4, dtype=jnp.float32), W4)
      @ (jnp.arange(128)[None, :] % 4
         == jnp.arange(4)[:, None]).astype(jnp.float32),
      jnp.broadcast_to(b4, (1, 128)))

    return out2d[:E // 128].reshape(E)
